# baseline (speedup 1.0000x reference)
import math

import jax
import jax.numpy as jnp
from jax import lax
from jax.experimental import pallas as pl
from jax.experimental.pallas import tpu as pltpu
from jax.experimental.pallas import tpu_sc as plsc

N = 10000
E = 160000
EMB = 8
HID = 16
NB = 8
RAD = 64
MAXR = 2.0
NN = 16.0

NC = 2
NS = 16
LANES = 16
NW = NC * NS
CHUNK = 128
NCH = 40
EPW = CHUNK * NCH
E_PAD = EPW * NW
N_PAD = 10240

EB = 4096
BN = 2000

_PREC = lax.Precision.DEFAULT


def _sc_mesh():
    return plsc.VectorSubcoreMesh(
        core_axis_name="c", subcore_axis_name="s", num_cores=NC, num_subcores=NS
    )


_SC_PARAMS = pltpu.CompilerParams(
    needs_layout_passes=False, use_tc_tiling_on_sc=False
)


def _worker_id():
    return lax.axis_index("s") * NC + lax.axis_index("c")


def _gather_chunks(hx_h, idx_v, rows_v, sem):
    def fire(j, carry):
        sl = pl.ds(j * CHUNK, CHUNK)
        pltpu.async_copy(hx_h.at[idx_v.at[sl]], rows_v.at[sl], sem)
        return carry

    lax.fori_loop(0, NCH, fire, 0)

    def drain(j, carry):
        sl = pl.ds(j * CHUNK, CHUNK)
        pltpu.make_async_copy(hx_h.at[idx_v.at[sl]], rows_v.at[sl], sem).wait()
        return carry

    lax.fori_loop(0, NCH, drain, 0)


def _sc_gather0_body(hx_h, src_h, dst_h, px_h, py_h, pz_h, hxs_h, r2_h,
                     idx_v, d_v, rows_v, px_v, py_v, pz_v, r2_v, sem):
    base = _worker_id() * EPW
    pltpu.sync_copy(src_h.at[pl.ds(base, EPW)], idx_v)
    pltpu.sync_copy(dst_h.at[pl.ds(base, EPW)], d_v)
    pltpu.sync_copy(px_h, px_v)
    pltpu.sync_copy(py_h, py_v)
    pltpu.sync_copy(pz_h, pz_v)

    def fire(j, carry):
        sl = pl.ds(j * CHUNK, CHUNK)
        pltpu.async_copy(hx_h.at[idx_v.at[sl]], rows_v.at[sl], sem)
        return carry

    lax.fori_loop(0, NCH, fire, 0)

    def body(i, carry):
        s16 = idx_v[pl.ds(i * LANES, LANES)]
        d16 = d_v[pl.ds(i * LANES, LANES)]
        dx = plsc.load_gather(px_v, [s16]) - plsc.load_gather(px_v, [d16])
        dy = plsc.load_gather(py_v, [s16]) - plsc.load_gather(py_v, [d16])
        dz = plsc.load_gather(pz_v, [s16]) - plsc.load_gather(pz_v, [d16])
        r2_v[pl.ds(i * LANES, LANES)] = dx * dx + dy * dy + dz * dz
        return carry

    lax.fori_loop(0, EPW // LANES, body, 0)
    pltpu.sync_copy(r2_v, r2_h.at[pl.ds(base, EPW)])

    def drain(j, carry):
        sl = pl.ds(j * CHUNK, CHUNK)
        pltpu.make_async_copy(hx_h.at[idx_v.at[sl]], rows_v.at[sl], sem).wait()
        return carry

    lax.fori_loop(0, NCH, drain, 0)
    pltpu.sync_copy(rows_v, hxs_h.at[pl.ds(base, EPW)])


def _sc_gather_body(hx_h, src_h, out_h, idx_v, rows_v, sem):
    base = _worker_id() * EPW
    pltpu.sync_copy(src_h.at[pl.ds(base, EPW)], idx_v)
    _gather_chunks(hx_h, idx_v, rows_v, sem)
    pltpu.sync_copy(rows_v, out_h.at[pl.ds(base, EPW)])


def _sc_scatter_body(msg_h, dst_h, zero_h, agg_h, didx_v, msg_v, agg_sh, sem):
    cid = lax.axis_index("c")
    sid = lax.axis_index("s")
    wid = sid * NC + cid
    base = wid * EPW
    pltpu.sync_copy(dst_h.at[wid], didx_v)
    pltpu.sync_copy(msg_h.at[pl.ds(base, EPW)], msg_v)
    zrows = N_PAD // NS
    pltpu.sync_copy(zero_h.at[pl.ds(sid * zrows, zrows)],
                    agg_sh.at[pl.ds(sid * zrows, zrows)])
    plsc.subcore_barrier()

    def fire(j, carry):
        pltpu.async_copy(msg_v.at[pl.ds(j * CHUNK, CHUNK)],
                         agg_sh.at[didx_v.at[j]], sem, add=True)
        return carry

    lax.fori_loop(0, NCH, fire, 0)

    def drain(j, carry):
        pltpu.make_async_copy(msg_v.at[pl.ds(j * CHUNK, CHUNK)],
                              agg_sh.at[didx_v.at[j]], sem).wait()
        return carry

    lax.fori_loop(0, NCH, drain, 0)
    plsc.subcore_barrier()
    orows = N // NS
    pltpu.sync_copy(agg_sh.at[pl.ds(sid * orows, orows)],
                    agg_h.at[cid].at[pl.ds(sid * orows, orows)])


def _sc_gather0(hx, src_p, dst_p, posx, posy, posz):
    return pl.kernel(
        _sc_gather0_body,
        out_type=(
            jax.ShapeDtypeStruct((E_PAD, HID), jnp.float32),
            jax.ShapeDtypeStruct((E_PAD,), jnp.float32),
        ),
        mesh=_sc_mesh(),
        compiler_params=_SC_PARAMS,
        scratch_types=[
            pltpu.VMEM((EPW,), jnp.int32),
            pltpu.VMEM((EPW,), jnp.int32),
            pltpu.VMEM((EPW, HID), jnp.float32),
            pltpu.VMEM((N,), jnp.float32),
            pltpu.VMEM((N,), jnp.float32),
            pltpu.VMEM((N,), jnp.float32),
            pltpu.VMEM((EPW,), jnp.float32),
            pltpu.SemaphoreType.DMA,
        ],
    )(hx, src_p, dst_p, posx, posy, posz)


def _sc_gather(hx, src_p):
    return pl.kernel(
        _sc_gather_body,
        out_type=jax.ShapeDtypeStruct((E_PAD, HID), jnp.float32),
        mesh=_sc_mesh(),
        compiler_params=_SC_PARAMS,
        scratch_types=[
            pltpu.VMEM((EPW,), jnp.int32),
            pltpu.VMEM((EPW, HID), jnp.float32),
            pltpu.SemaphoreType.DMA,
        ],
    )(hx, src_p)


def _sc_scatter(msg, dst3, zeros_pad):
    return pl.kernel(
        _sc_scatter_body,
        out_type=jax.ShapeDtypeStruct((NC, N, HID), jnp.float32),
        mesh=_sc_mesh(),
        compiler_params=_SC_PARAMS,
        scratch_types=[
            pltpu.VMEM((NCH, CHUNK), jnp.int32),
            pltpu.VMEM((EPW, HID), jnp.float32),
            pltpu.VMEM_SHARED((N_PAD, HID), jnp.float32),
            pltpu.SemaphoreType.DMA,
        ],
    )(msg, dst3, zeros_pad)


def _rep_mat(k, m):
    jj = lax.broadcasted_iota(jnp.int32, (k, k * m), 1)
    ii = lax.broadcasted_iota(jnp.int32, (k, k * m), 0)
    return (jj // m == ii).astype(jnp.float32)


def _tile_mat(m, k):
    jj = lax.broadcasted_iota(jnp.int32, (m, k * m), 1)
    ii = lax.broadcasted_iota(jnp.int32, (m, k * m), 0)
    return (jj % m == ii).astype(jnp.float32)


def _sum_mat(k, m):
    ii = lax.broadcasted_iota(jnp.int32, (k * m, m), 0)
    jj = lax.broadcasted_iota(jnp.int32, (k * m, m), 1)
    return (ii % m == jj).astype(jnp.float32)


def _outer_sq(h, k):
    rep = jnp.dot(h, _rep_mat(k, k), precision=_PREC)
    til = jnp.dot(h, _tile_mat(k, k), precision=_PREC)
    return rep * til


def _tc_init_body(x_ref, z_ref, emb_ref, wtp_ref, wsc_ref, wlin_ref,
                  sc_ref, hxp_ref):
    xi = x_ref[...]
    oh = (lax.broadcasted_iota(jnp.int32, (BN, 20), 1) == xi)
    h = jnp.dot(oh.astype(jnp.float32), emb_ref[...], precision=_PREC)
    hh = _outer_sq(h, EMB)
    h0 = jnp.dot(hh, wtp_ref[...], precision=_PREC) * (1.0 / EMB)
    sc_ref[...] = (jnp.dot(h0, wsc_ref[...], precision=_PREC)
                   * (1.0 / math.sqrt(EMB)) * z_ref[...])
    hx = jnp.dot(h0, wlin_ref[...], precision=_PREC) * (1.0 / math.sqrt(EMB))
    hxp_ref[...] = jnp.concatenate(
        [hx, jnp.zeros((BN, HID - EMB), jnp.float32)], axis=1)


def _tc_edge_body(ind, r2_ref, hxs_ref, w1_ref, w2_ref, msg_ref):
    r2 = r2_ref[...]
    r = jnp.sqrt(r2 + 1e-12)
    nv = (lax.broadcasted_iota(jnp.int32, (NB, 1), 0).astype(jnp.float32)
          + 1.0) * (math.pi / MAXR)
    ebT = (jnp.sin(nv * r) * (math.sqrt(2.0 / MAXR) * math.sqrt(float(NB)))
           / (r + 1e-9))
    u = 2.0 * (r / MAXR - 1.0)
    ea = (1.0 - jnp.cos(math.pi * u)) * 0.5
    ea = jnp.where(u > 0, 0.0, ea)
    ea = jnp.where(u < -1.0, 1.0, ea)
    z1T = lax.dot_general(w1_ref[...], ebT, (((0,), (0,)), ((), ())),
                          precision=_PREC) * (1.0 / math.sqrt(NB))
    sT = z1T * jax.nn.sigmoid(z1T) * ea
    hxsP = hxs_ref[...]
    qb = EB // 8
    wscale = 1.0 / (math.sqrt(RAD) * math.sqrt(ind))
    jj = lax.broadcasted_iota(jnp.int32, (HID, ind * HID), 1)
    ii = lax.broadcasted_iota(jnp.int32, (HID, ind * HID), 0)
    rep = (jj // HID == ii).astype(jnp.float32)
    summ = _sum_mat(ind, HID)
    parts = []
    for p in range(8):
        hxs_p = hxsP[:, p * HID:(p + 1) * HID]
        hxrep_p = jnp.dot(hxs_p, rep, precision=_PREC)
        w_p = lax.dot_general(sT[:, p * qb:(p + 1) * qb], w2_ref[...],
                              (((0,), (0,)), ((), ())),
                              precision=_PREC) * wscale
        parts.append(jnp.dot(w_p * hxrep_p, summ, precision=_PREC))
    msg_ref[...] = jnp.concatenate(parts, axis=1)


def _tc_update_body(sc_ref, a0_ref, a1_ref, wlin2_ref, wsc_ref, wlin1_ref,
                    z_ref, sc1_ref, hx1_ref):
    agg = (a0_ref[...] + a1_ref[...]) * (1.0 / math.sqrt(NN))
    y = sc_ref[...] + jnp.dot(agg, wlin2_ref[...], precision=_PREC) * (
        1.0 / math.sqrt(HID))
    y = y * lax.rsqrt(jnp.mean(y * y, axis=1, keepdims=True) + 1e-6)
    h = y * jax.nn.sigmoid(y)
    sc1_ref[...] = (jnp.dot(h, wsc_ref[...], precision=_PREC)
                    * (1.0 / math.sqrt(HID)) * z_ref[...])
    hx1_ref[...] = jnp.dot(h, wlin1_ref[...], precision=_PREC) * (
        1.0 / math.sqrt(HID))


def _tc_final_body(sc_ref, a0_ref, a1_ref, wlin2_ref, wa_ref, wb_ref, out_ref):
    agg = (a0_ref[...] + a1_ref[...]) * (1.0 / math.sqrt(NN))
    y = sc_ref[...] + jnp.dot(agg, wlin2_ref[...], precision=_PREC) * (
        1.0 / math.sqrt(HID))
    y = y * lax.rsqrt(jnp.mean(y * y, axis=1, keepdims=True) + 1e-6)
    h = y * jax.nn.sigmoid(y)
    hh = _outer_sq(h, HID)
    za = jnp.dot(hh, wa_ref[...], precision=_PREC) * (1.0 / HID)
    ha = za * jax.nn.sigmoid(za)
    hha = _outer_sq(ha, HID)
    hb = jnp.dot(hha, wb_ref[...], precision=_PREC) * (1.0 / HID)
    part = jnp.sum(hb) * (1.0 / math.sqrt(float(N)))

    @pl.when(pl.program_id(0) == 0)
    def _():
        out_ref[...] = jnp.zeros((1, 1), jnp.float32)

    out_ref[...] += part


def _full(shape):
    return pl.BlockSpec(shape, lambda i: (0,) * len(shape))


def _tc_init(xi, z, emb, wtp, wsc, wlin):
    grid = (N // BN,)
    return pl.pallas_call(
        _tc_init_body,
        grid=grid,
        in_specs=[
            pl.BlockSpec((BN, 1), lambda i: (i, 0)),
            pl.BlockSpec((BN, 1), lambda i: (i, 0)),
            _full((20, EMB)),
            _full((EMB * EMB, EMB)),
            _full((EMB, HID)),
            _full((EMB, EMB)),
        ],
        out_specs=[
            pl.BlockSpec((BN, HID), lambda i: (i, 0)),
            pl.BlockSpec((BN, HID), lambda i: (i, 0)),
        ],
        out_shape=[
            jax.ShapeDtypeStruct((N, HID), jnp.float32),
            jax.ShapeDtypeStruct((N, HID), jnp.float32),
        ],
    )(xi, z, emb, wtp, wsc, wlin)


def _tc_edge(ind, r2p, hxsP, w1, w2):
    grid = (E_PAD // EB,)
    body = lambda *refs: _tc_edge_body(ind, *refs)
    return pl.pallas_call(
        body,
        grid=grid,
        in_specs=[
            pl.BlockSpec((1, EB), lambda i: (0, i)),
            pl.BlockSpec((EB // 8, 128), lambda i: (i, 0)),
            _full((NB, RAD)),
            _full((RAD, ind * HID)),
        ],
        out_specs=pl.BlockSpec((EB // 8, 128), lambda i: (i, 0)),
        out_shape=jax.ShapeDtypeStruct((E_PAD // 8, 128), jnp.float32),
    )(r2p, hxsP, w1, w2)


def _tc_update(sc, a0, a1, wlin2, wsc, wlin1, z):
    grid = (N // BN,)
    nspec = pl.BlockSpec((BN, HID), lambda i: (i, 0))
    return pl.pallas_call(
        _tc_update_body,
        grid=grid,
        in_specs=[nspec, nspec, nspec, _full((HID, HID)), _full((HID, HID)),
                  _full((HID, HID)), pl.BlockSpec((BN, 1), lambda i: (i, 0))],
        out_specs=[nspec, nspec],
        out_shape=[
            jax.ShapeDtypeStruct((N, HID), jnp.float32),
            jax.ShapeDtypeStruct((N, HID), jnp.float32),
        ],
    )(sc, a0, a1, wlin2, wsc, wlin1, z)


def _tc_final(sc, a0, a1, wlin2, wa, wb):
    grid = (N // BN,)
    nspec = pl.BlockSpec((BN, HID), lambda i: (i, 0))
    return pl.pallas_call(
        _tc_final_body,
        grid=grid,
        in_specs=[nspec, nspec, nspec, _full((HID, HID)),
                  _full((HID * HID, HID)), _full((HID * HID, 1))],
        out_specs=pl.BlockSpec((1, 1), lambda i: (0, 0)),
        out_shape=jax.ShapeDtypeStruct((1, 1), jnp.float32),
    )(sc, a0, a1, wlin2, wa, wb)


def kernel(pos, x, z, edge_index, batch, emb, W_tp0, Wsc0, Wlin1_0, Wfc1_0,
           Wfc2_0, Wlin2_0, Wsc1, Wlin1_1, Wfc1_1, Wfc2_1, Wlin2_1, W_a, W_b):
    src = edge_index[0].astype(jnp.int32)
    dst = edge_index[1].astype(jnp.int32)
    padn = E_PAD - E
    src_p = jnp.concatenate([src, jnp.zeros((padn,), jnp.int32)])
    dst_p = jnp.concatenate([dst, jnp.full((padn,), N, jnp.int32)])
    dst3 = dst_p.reshape(NW, NCH, CHUNK)
    posx = pos[:, 0]
    posy = pos[:, 1]
    posz = pos[:, 2]
    zeros_pad = jnp.zeros((N_PAD, HID), jnp.float32)
    wtp = W_tp0.reshape(EMB * EMB, EMB)
    wa = W_a.reshape(HID * HID, HID)
    wb = W_b.reshape(HID * HID, 1)
    xi = x.astype(jnp.int32)

    sc0, hxp0 = _tc_init(xi, z, emb, wtp, Wsc0, Wlin1_0)
    hxs0, r2 = _sc_gather0(hxp0, src_p, dst_p, posx, posy, posz)
    r2p = (r2.reshape(E_PAD // EB, EB // 8, 8)
           .transpose(0, 2, 1).reshape(1, E_PAD))
    msg0 = _tc_edge(EMB, r2p, hxs0.reshape(E_PAD // 8, 128), Wfc1_0, Wfc2_0)
    agg0 = _sc_scatter(msg0.reshape(E_PAD, HID), dst3, zeros_pad)
    sc1, hx1 = _tc_update(sc0, agg0[0], agg0[1], Wlin2_0, Wsc1, Wlin1_1, z)
    hxs1 = _sc_gather(hx1, src_p)
    msg1 = _tc_edge(HID, r2p, hxs1.reshape(E_PAD // 8, 128), Wfc1_1, Wfc2_1)
    agg1 = _sc_scatter(msg1.reshape(E_PAD, HID), dst3, zeros_pad)
    out = _tc_final(sc1, agg1[0], agg1[1], Wlin2_1, wa, wb)
    return out


# speedup vs baseline: 1.0619x; 1.0619x over previous
import math

import jax
import jax.numpy as jnp
from jax import lax
from jax.experimental import pallas as pl
from jax.experimental.pallas import tpu as pltpu
from jax.experimental.pallas import tpu_sc as plsc

N = 10000
E = 160000
EMB = 8
HID = 16
NB = 8
RAD = 64
MAXR = 2.0
NN = 16.0

NC = 2
NS = 16
LANES = 16
NW = NC * NS
CHUNK = 128
NCH = 40
EPW = CHUNK * NCH
E_PAD = EPW * NW
N_PAD = 10240

EB = 8192
BN = 2000

_PREC = lax.Precision.DEFAULT


def _sc_mesh():
    return plsc.VectorSubcoreMesh(
        core_axis_name="c", subcore_axis_name="s", num_cores=NC, num_subcores=NS
    )


_SC_PARAMS = pltpu.CompilerParams(
    needs_layout_passes=False, use_tc_tiling_on_sc=False
)


def _worker_id():
    return lax.axis_index("s") * NC + lax.axis_index("c")


def _gather_chunks(hx_h, idx_v, rows_v, sem):
    def fire(j, carry):
        sl = pl.ds(j * CHUNK, CHUNK)
        pltpu.async_copy(hx_h.at[idx_v.at[sl]], rows_v.at[sl], sem)
        return carry

    lax.fori_loop(0, NCH, fire, 0)

    def drain(j, carry):
        sl = pl.ds(j * CHUNK, CHUNK)
        pltpu.make_async_copy(hx_h.at[idx_v.at[sl]], rows_v.at[sl], sem).wait()
        return carry

    lax.fori_loop(0, NCH, drain, 0)


def _sc_gather0_body(hx_h, src_h, dst_h, px_h, py_h, pz_h, hxs_h, r2_h,
                     idx_v, d_v, rows_v, px_v, py_v, pz_v, r2_v, sem):
    base = _worker_id() * EPW
    pltpu.sync_copy(src_h.at[pl.ds(base, EPW)], idx_v)
    pltpu.sync_copy(dst_h.at[pl.ds(base, EPW)], d_v)
    pltpu.sync_copy(px_h, px_v)
    pltpu.sync_copy(py_h, py_v)
    pltpu.sync_copy(pz_h, pz_v)

    def fire(j, carry):
        sl = pl.ds(j * CHUNK, CHUNK)
        pltpu.async_copy(hx_h.at[idx_v.at[sl]], rows_v.at[sl], sem)
        return carry

    lax.fori_loop(0, NCH, fire, 0)

    def body(i, carry):
        s16 = idx_v[pl.ds(i * LANES, LANES)]
        d16 = d_v[pl.ds(i * LANES, LANES)]
        dx = plsc.load_gather(px_v, [s16]) - plsc.load_gather(px_v, [d16])
        dy = plsc.load_gather(py_v, [s16]) - plsc.load_gather(py_v, [d16])
        dz = plsc.load_gather(pz_v, [s16]) - plsc.load_gather(pz_v, [d16])
        r2_v[pl.ds(i * LANES, LANES)] = dx * dx + dy * dy + dz * dz
        return carry

    lax.fori_loop(0, EPW // LANES, body, 0)
    pltpu.sync_copy(r2_v, r2_h.at[pl.ds(base, EPW)])

    def drain(j, carry):
        sl = pl.ds(j * CHUNK, CHUNK)
        pltpu.make_async_copy(hx_h.at[idx_v.at[sl]], rows_v.at[sl], sem).wait()
        return carry

    lax.fori_loop(0, NCH, drain, 0)
    pltpu.sync_copy(rows_v, hxs_h.at[pl.ds(base, EPW)])


def _sc_gather_body(hx_h, src_h, out_h, idx_v, rows_v, sem):
    base = _worker_id() * EPW
    pltpu.sync_copy(src_h.at[pl.ds(base, EPW)], idx_v)
    _gather_chunks(hx_h, idx_v, rows_v, sem)
    pltpu.sync_copy(rows_v, out_h.at[pl.ds(base, EPW)])


def _sc_scatter_body(msg_h, dst_h, zero_h, agg_h, didx_v, msg_v, agg_sh, sem):
    cid = lax.axis_index("c")
    sid = lax.axis_index("s")
    wid = sid * NC + cid
    base = wid * EPW
    pltpu.sync_copy(dst_h.at[wid], didx_v)
    pltpu.sync_copy(msg_h.at[pl.ds(base, EPW)], msg_v)
    zrows = N_PAD // NS
    pltpu.sync_copy(zero_h.at[pl.ds(sid * zrows, zrows)],
                    agg_sh.at[pl.ds(sid * zrows, zrows)])
    plsc.subcore_barrier()

    def fire(j, carry):
        pltpu.async_copy(msg_v.at[pl.ds(j * CHUNK, CHUNK)],
                         agg_sh.at[didx_v.at[j]], sem, add=True)
        return carry

    lax.fori_loop(0, NCH, fire, 0)

    def drain(j, carry):
        pltpu.make_async_copy(msg_v.at[pl.ds(j * CHUNK, CHUNK)],
                              agg_sh.at[didx_v.at[j]], sem).wait()
        return carry

    lax.fori_loop(0, NCH, drain, 0)
    plsc.subcore_barrier()
    orows = N // NS
    pltpu.sync_copy(agg_sh.at[pl.ds(sid * orows, orows)],
                    agg_h.at[cid].at[pl.ds(sid * orows, orows)])


def _sc_gather0(hx, src_p, dst_p, posx, posy, posz):
    return pl.kernel(
        _sc_gather0_body,
        out_type=(
            jax.ShapeDtypeStruct((E_PAD, HID), jnp.float32),
            jax.ShapeDtypeStruct((E_PAD,), jnp.float32),
        ),
        mesh=_sc_mesh(),
        compiler_params=_SC_PARAMS,
        scratch_types=[
            pltpu.VMEM((EPW,), jnp.int32),
            pltpu.VMEM((EPW,), jnp.int32),
            pltpu.VMEM((EPW, HID), jnp.float32),
            pltpu.VMEM((N,), jnp.float32),
            pltpu.VMEM((N,), jnp.float32),
            pltpu.VMEM((N,), jnp.float32),
            pltpu.VMEM((EPW,), jnp.float32),
            pltpu.SemaphoreType.DMA,
        ],
    )(hx, src_p, dst_p, posx, posy, posz)


def _sc_gather(hx, src_p):
    return pl.kernel(
        _sc_gather_body,
        out_type=jax.ShapeDtypeStruct((E_PAD, HID), jnp.float32),
        mesh=_sc_mesh(),
        compiler_params=_SC_PARAMS,
        scratch_types=[
            pltpu.VMEM((EPW,), jnp.int32),
            pltpu.VMEM((EPW, HID), jnp.float32),
            pltpu.SemaphoreType.DMA,
        ],
    )(hx, src_p)


def _sc_scatter(msg, dst3, zeros_pad):
    return pl.kernel(
        _sc_scatter_body,
        out_type=jax.ShapeDtypeStruct((NC, N, HID), jnp.float32),
        mesh=_sc_mesh(),
        compiler_params=_SC_PARAMS,
        scratch_types=[
            pltpu.VMEM((NCH, CHUNK), jnp.int32),
            pltpu.VMEM((EPW, HID), jnp.float32),
            pltpu.VMEM_SHARED((N_PAD, HID), jnp.float32),
            pltpu.SemaphoreType.DMA,
        ],
    )(msg, dst3, zeros_pad)


def _rep_mat(k, m):
    jj = lax.broadcasted_iota(jnp.int32, (k, k * m), 1)
    ii = lax.broadcasted_iota(jnp.int32, (k, k * m), 0)
    return (jj // m == ii).astype(jnp.float32)


def _tile_mat(m, k):
    jj = lax.broadcasted_iota(jnp.int32, (m, k * m), 1)
    ii = lax.broadcasted_iota(jnp.int32, (m, k * m), 0)
    return (jj % m == ii).astype(jnp.float32)


def _sum_mat(k, m):
    ii = lax.broadcasted_iota(jnp.int32, (k * m, m), 0)
    jj = lax.broadcasted_iota(jnp.int32, (k * m, m), 1)
    return (ii % m == jj).astype(jnp.float32)


def _outer_sq(h, k):
    rep = jnp.dot(h, _rep_mat(k, k), precision=_PREC)
    til = jnp.dot(h, _tile_mat(k, k), precision=_PREC)
    return rep * til


def _tc_init_body(x_ref, z_ref, emb_ref, wtp_ref, wsc_ref, wlin_ref,
                  sc_ref, hxp_ref):
    xi = x_ref[...]
    oh = (lax.broadcasted_iota(jnp.int32, (BN, 20), 1) == xi)
    h = jnp.dot(oh.astype(jnp.float32), emb_ref[...], precision=_PREC)
    hh = _outer_sq(h, EMB)
    h0 = jnp.dot(hh, wtp_ref[...], precision=_PREC) * (1.0 / EMB)
    sc_ref[...] = (jnp.dot(h0, wsc_ref[...], precision=_PREC)
                   * (1.0 / math.sqrt(EMB)) * z_ref[...])
    hx = jnp.dot(h0, wlin_ref[...], precision=_PREC) * (1.0 / math.sqrt(EMB))
    hxp_ref[...] = jnp.concatenate(
        [hx, jnp.zeros((BN, HID - EMB), jnp.float32)], axis=1)


def _tc_edge_body(ind, r2_ref, hxs_ref, w1_ref, w2_ref, msg_ref):
    r2 = r2_ref[...]
    r = jnp.sqrt(r2 + 1e-12)
    nv = (lax.broadcasted_iota(jnp.int32, (NB, 1), 0).astype(jnp.float32)
          + 1.0) * (math.pi / MAXR)
    ebT = (jnp.sin(nv * r) * (math.sqrt(2.0 / MAXR) * math.sqrt(float(NB)))
           / (r + 1e-9))
    u = 2.0 * (r / MAXR - 1.0)
    ea = (1.0 - jnp.cos(math.pi * u)) * 0.5
    ea = jnp.where(u > 0, 0.0, ea)
    ea = jnp.where(u < -1.0, 1.0, ea)
    z1T = lax.dot_general(w1_ref[...], ebT, (((0,), (0,)), ((), ())),
                          precision=_PREC) * (1.0 / math.sqrt(NB))
    sT = z1T * jax.nn.sigmoid(z1T) * ea
    hxsP = hxs_ref[...]
    qb = EB // 8
    wscale = 1.0 / (math.sqrt(RAD) * math.sqrt(ind))
    jj = lax.broadcasted_iota(jnp.int32, (HID, ind * HID), 1)
    ii = lax.broadcasted_iota(jnp.int32, (HID, ind * HID), 0)
    rep = (jj // HID == ii).astype(jnp.float32)
    summ = _sum_mat(ind, HID)
    parts = []
    for p in range(8):
        hxs_p = hxsP[:, p * HID:(p + 1) * HID]
        hxrep_p = jnp.dot(hxs_p, rep, precision=_PREC)
        w_p = lax.dot_general(sT[:, p * qb:(p + 1) * qb], w2_ref[...],
                              (((0,), (0,)), ((), ())),
                              precision=_PREC) * wscale
        parts.append(jnp.dot(w_p * hxrep_p, summ, precision=_PREC))
    msg_ref[...] = jnp.concatenate(parts, axis=1)


def _tc_update_body(sc_ref, a0_ref, a1_ref, wlin2_ref, wsc_ref, wlin1_ref,
                    z_ref, sc1_ref, hx1_ref):
    agg = (a0_ref[...] + a1_ref[...]) * (1.0 / math.sqrt(NN))
    y = sc_ref[...] + jnp.dot(agg, wlin2_ref[...], precision=_PREC) * (
        1.0 / math.sqrt(HID))
    y = y * lax.rsqrt(jnp.mean(y * y, axis=1, keepdims=True) + 1e-6)
    h = y * jax.nn.sigmoid(y)
    sc1_ref[...] = (jnp.dot(h, wsc_ref[...], precision=_PREC)
                    * (1.0 / math.sqrt(HID)) * z_ref[...])
    hx1_ref[...] = jnp.dot(h, wlin1_ref[...], precision=_PREC) * (
        1.0 / math.sqrt(HID))


def _tc_final_body(sc_ref, a0_ref, a1_ref, wlin2_ref, wa_ref, wb_ref, out_ref):
    agg = (a0_ref[...] + a1_ref[...]) * (1.0 / math.sqrt(NN))
    y = sc_ref[...] + jnp.dot(agg, wlin2_ref[...], precision=_PREC) * (
        1.0 / math.sqrt(HID))
    y = y * lax.rsqrt(jnp.mean(y * y, axis=1, keepdims=True) + 1e-6)
    h = y * jax.nn.sigmoid(y)
    hh = _outer_sq(h, HID)
    za = jnp.dot(hh, wa_ref[...], precision=_PREC) * (1.0 / HID)
    ha = za * jax.nn.sigmoid(za)
    hha = _outer_sq(ha, HID)
    hb = jnp.dot(hha, wb_ref[...], precision=_PREC) * (1.0 / HID)
    part = jnp.sum(hb) * (1.0 / math.sqrt(float(N)))

    @pl.when(pl.program_id(0) == 0)
    def _():
        out_ref[...] = jnp.zeros((1, 1), jnp.float32)

    out_ref[...] += part


def _full(shape):
    return pl.BlockSpec(shape, lambda i: (0,) * len(shape))


def _tc_init(xi, z, emb, wtp, wsc, wlin):
    grid = (N // BN,)
    return pl.pallas_call(
        _tc_init_body,
        grid=grid,
        in_specs=[
            pl.BlockSpec((BN, 1), lambda i: (i, 0)),
            pl.BlockSpec((BN, 1), lambda i: (i, 0)),
            _full((20, EMB)),
            _full((EMB * EMB, EMB)),
            _full((EMB, HID)),
            _full((EMB, EMB)),
        ],
        out_specs=[
            pl.BlockSpec((BN, HID), lambda i: (i, 0)),
            pl.BlockSpec((BN, HID), lambda i: (i, 0)),
        ],
        out_shape=[
            jax.ShapeDtypeStruct((N, HID), jnp.float32),
            jax.ShapeDtypeStruct((N, HID), jnp.float32),
        ],
    )(xi, z, emb, wtp, wsc, wlin)


def _tc_edge(ind, r2p, hxsP, w1, w2):
    grid = (E_PAD // EB,)
    body = lambda *refs: _tc_edge_body(ind, *refs)
    return pl.pallas_call(
        body,
        grid=grid,
        in_specs=[
            pl.BlockSpec((1, EB), lambda i: (0, i)),
            pl.BlockSpec((EB // 8, 128), lambda i: (i, 0)),
            _full((NB, RAD)),
            _full((RAD, ind * HID)),
        ],
        out_specs=pl.BlockSpec((EB // 8, 128), lambda i: (i, 0)),
        out_shape=jax.ShapeDtypeStruct((E_PAD // 8, 128), jnp.float32),
    )(r2p, hxsP, w1, w2)


def _tc_update(sc, a0, a1, wlin2, wsc, wlin1, z):
    grid = (N // BN,)
    nspec = pl.BlockSpec((BN, HID), lambda i: (i, 0))
    return pl.pallas_call(
        _tc_update_body,
        grid=grid,
        in_specs=[nspec, nspec, nspec, _full((HID, HID)), _full((HID, HID)),
                  _full((HID, HID)), pl.BlockSpec((BN, 1), lambda i: (i, 0))],
        out_specs=[nspec, nspec],
        out_shape=[
            jax.ShapeDtypeStruct((N, HID), jnp.float32),
            jax.ShapeDtypeStruct((N, HID), jnp.float32),
        ],
    )(sc, a0, a1, wlin2, wsc, wlin1, z)


def _tc_final(sc, a0, a1, wlin2, wa, wb):
    grid = (N // BN,)
    nspec = pl.BlockSpec((BN, HID), lambda i: (i, 0))
    return pl.pallas_call(
        _tc_final_body,
        grid=grid,
        in_specs=[nspec, nspec, nspec, _full((HID, HID)),
                  _full((HID * HID, HID)), _full((HID * HID, 1))],
        out_specs=pl.BlockSpec((1, 1), lambda i: (0, 0)),
        out_shape=jax.ShapeDtypeStruct((1, 1), jnp.float32),
    )(sc, a0, a1, wlin2, wa, wb)


def kernel(pos, x, z, edge_index, batch, emb, W_tp0, Wsc0, Wlin1_0, Wfc1_0,
           Wfc2_0, Wlin2_0, Wsc1, Wlin1_1, Wfc1_1, Wfc2_1, Wlin2_1, W_a, W_b):
    src = edge_index[0].astype(jnp.int32)
    dst = edge_index[1].astype(jnp.int32)
    padn = E_PAD - E
    src_p = jnp.concatenate([src, jnp.zeros((padn,), jnp.int32)])
    dst_p = jnp.concatenate([dst, jnp.full((padn,), N, jnp.int32)])
    dst3 = dst_p.reshape(NW, NCH, CHUNK)
    posx = pos[:, 0]
    posy = pos[:, 1]
    posz = pos[:, 2]
    zeros_pad = jnp.zeros((N_PAD, HID), jnp.float32)
    wtp = W_tp0.reshape(EMB * EMB, EMB)
    wa = W_a.reshape(HID * HID, HID)
    wb = W_b.reshape(HID * HID, 1)
    xi = x.astype(jnp.int32)

    sc0, hxp0 = _tc_init(xi, z, emb, wtp, Wsc0, Wlin1_0)
    hxs0, r2 = _sc_gather0(hxp0, src_p, dst_p, posx, posy, posz)
    r2p = (r2.reshape(E_PAD // EB, EB // 8, 8)
           .transpose(0, 2, 1).reshape(1, E_PAD))
    msg0 = _tc_edge(EMB, r2p, hxs0.reshape(E_PAD // 8, 128), Wfc1_0, Wfc2_0)
    agg0 = _sc_scatter(msg0.reshape(E_PAD, HID), dst3, zeros_pad)
    sc1, hx1 = _tc_update(sc0, agg0[0], agg0[1], Wlin2_0, Wsc1, Wlin1_1, z)
    hxs1 = _sc_gather(hx1, src_p)
    msg1 = _tc_edge(HID, r2p, hxs1.reshape(E_PAD // 8, 128), Wfc1_1, Wfc2_1)
    agg1 = _sc_scatter(msg1.reshape(E_PAD, HID), dst3, zeros_pad)
    out = _tc_final(sc1, agg1[0], agg1[1], Wlin2_1, wa, wb)
    return out
